# Initial kernel scaffold; baseline (speedup 1.0000x reference)
import jax
import jax.numpy as jnp
from jax.experimental import pallas as pl


def kernel(x, pos_table):
    raise NotImplementedError("write your pallas kernel here")


# speedup vs baseline: 3.2808x; 3.2808x over previous
import jax
import jax.numpy as jnp
from jax.experimental import pallas as pl


S_BLK = 512


def _pos_add_kernel(x_ref, pos_ref, out_ref):
    out_ref[...] = x_ref[...] + pos_ref[...][None, :, :]


def kernel(x, pos_table):
    batch, seq_len, d_model = x.shape
    n_blocks = seq_len // S_BLK
    return pl.pallas_call(
        _pos_add_kernel,
        grid=(n_blocks,),
        in_specs=[
            pl.BlockSpec((batch, S_BLK, d_model), lambda s: (0, s, 0)),
            pl.BlockSpec((S_BLK, d_model), lambda s: (s, 0)),
        ],
        out_specs=pl.BlockSpec((batch, S_BLK, d_model), lambda s: (0, s, 0)),
        out_shape=jax.ShapeDtypeStruct((batch, seq_len, d_model), x.dtype),
    )(x, pos_table[:seq_len])


# speedup vs baseline: 3.2826x; 1.0005x over previous
import jax
import jax.numpy as jnp
from jax.experimental import pallas as pl


S_BLK = 256


def _pos_add_kernel(x_ref, pos_ref, out_ref):
    out_ref[...] = x_ref[...] + pos_ref[...][None, :, :]


def kernel(x, pos_table):
    batch, seq_len, d_model = x.shape
    n_blocks = seq_len // S_BLK
    return pl.pallas_call(
        _pos_add_kernel,
        grid=(n_blocks,),
        in_specs=[
            pl.BlockSpec((batch, S_BLK, d_model), lambda s: (0, s, 0)),
            pl.BlockSpec((S_BLK, d_model), lambda s: (s, 0)),
        ],
        out_specs=pl.BlockSpec((batch, S_BLK, d_model), lambda s: (0, s, 0)),
        out_shape=jax.ShapeDtypeStruct((batch, seq_len, d_model), x.dtype),
    )(x, pos_table[:seq_len])
